# baseline (speedup 1.0000x reference)
import functools

import jax
import jax.numpy as jnp
from jax import lax
from jax.experimental import pallas as pl
from jax.experimental.pallas import tpu as pltpu
from jax.experimental.pallas import tpu_sc as plsc

F32 = jnp.float32


def _pick_block(n, target):
    d = min(n, target)
    while n % d:
        d -= 1
    return d


def _softplus(z):
    return jnp.maximum(z, 0.0) + jnp.log1p(jnp.exp(-jnp.abs(z)))


def _proj_body(x_ref, w_ref, b_ref, out_ref):
    xb = x_ref[...]
    for h in range(2):
        out_ref[h] = (
            lax.dot_general(xb, w_ref[h], (((1,), (1,)), ((), ())),
                            preferred_element_type=F32)
            + b_ref[h]
        )


def _project_halved(a, W, b, block_rows):
    R, D = a.shape
    H = D // 2
    grid = (R // block_rows,)
    W2 = W.reshape(2, H, D)
    b2 = b.reshape(2, H)
    return pl.pallas_call(
        _proj_body,
        grid=grid,
        in_specs=[
            pl.BlockSpec((block_rows, D), lambda i: (i, 0)),
            pl.BlockSpec((2, H, D), lambda i: (0, 0, 0)),
            pl.BlockSpec((2, H), lambda i: (0, 0)),
        ],
        out_specs=pl.BlockSpec((2, block_rows, H), lambda i: (0, i, 0)),
        out_shape=jax.ShapeDtypeStruct((2, R, H), F32),
    )(a, W2, b2)


def _sc_edge_body(row_h, col_h, xs_h, xd_h, xu_h, ea_h,
                  m_h, hsum_h, ssum_h, statp_h,
                  h_acc, s_acc,
                  ridx0, cidx0, cgidx0, xsb0, xdb0, xub0, eab0,
                  ridx1, cidx1, cgidx1, xsb1, xdb1, xub1, eab1,
                  statb, gsem0,
                  *, N, E, C):
    ridx = (ridx0, ridx1)
    cidx = (cidx0, cidx1)
    cgidx = (cgidx0, cgidx1)
    xsb = (xsb0, xsb1)
    xdb = (xdb0, xdb1)
    xub = (xub0, xub1)
    eab = (eab0, eab1)
    mb = (xdb0, xdb1)
    c = lax.axis_index("c")
    s = lax.axis_index("s")
    num_chunks = E // C

    q, r = divmod(N // 8, 16)
    counts = sorted({8 * q} | ({8 * (q + 1)} if r else set()))
    base0 = 8 * (q * s + jnp.minimum(s, r))

    def _zero_row(e, _):
        for j in range(4):
            xdb0[e, pl.ds(j * 16, 16)] = jnp.zeros((16,), F32)
        return 0
    lax.fori_loop(0, C, _zero_row, 0)
    for j in range(4):
        statb[0, pl.ds(j * 16, 16)] = jnp.zeros((16,), F32)
        statb[1, pl.ds(j * 16, 16)] = jnp.zeros((16,), F32)
    rows_per_sub = 8 * (q + (s < r).astype(jnp.int32))
    fl = min(120, C)
    for cnt in counts:
        if cnt == 0:
            continue

        @pl.when(rows_per_sub == cnt)
        def _(cnt=cnt):
            off = 0
            while off < cnt:
                n = min(fl, cnt - off)
                pltpu.sync_copy(xdb0.at[pl.ds(0, n)],
                                h_acc.at[pl.ds(base0 + off, n)])
                pltpu.sync_copy(xdb0.at[pl.ds(0, n)],
                                s_acc.at[pl.ds(base0 + off, n)])
                off += n
    plsc.subcore_barrier()

    goff = c * N
    n_s = (num_chunks - s + 15) // 16

    def _fire_in(b, i):
        base = (s + i * 16) * C
        pltpu.sync_copy(row_h.at[pl.ds(base, C)], ridx[b])
        pltpu.sync_copy(col_h.at[pl.ds(base, C)], cidx[b])
        for j in range(C // 16):
            sl = pl.ds(j * 16, 16)
            ridx[b][sl] = ridx[b][sl] + goff
            cgidx[b][sl] = cidx[b][sl] + goff
        return (
            pltpu.async_copy(xs_h.at[ridx[b]], xsb[b], gsem0),
            pltpu.async_copy(xd_h.at[cgidx[b]], xdb[b], gsem0),
            pltpu.async_copy(xu_h.at[cgidx[b]], xub[b], gsem0),
            pltpu.async_copy(ea_h.at[pl.ds(c * E + base, C)], eab[b], gsem0),
        )

    def _compute(b):
        def _edge(e, _):
            for j in range(4):
                sl = pl.ds(j * 16, 16)
                mv = xsb[b][e, sl] + xdb[b][e, sl] + eab[b][e, sl]
                mb[b][e, sl] = mv
                statb[0, sl] += mv
                statb[1, sl] += mv * mv
                sg = 1.0 / (1.0 + jnp.exp(-mv))
                xsb[b][e, sl] = sg
                xub[b][e, sl] = xub[b][e, sl] * sg
            return 0
        lax.fori_loop(0, C, _edge, 0)

    def _out(b, i):
        base = (s + i * 16) * C
        pltpu.sync_copy(mb[b], m_h.at[pl.ds(c * E + base, C)])
        pltpu.sync_copy(xsb[b], s_acc.at[cidx[b]], add=True)
        pltpu.sync_copy(xub[b], h_acc.at[cidx[b]], add=True)

    @pl.when(n_s > 0)
    def _():
        ds0 = _fire_in(0, 0)
        for d in ds0:
            d.wait()

    def _step(step, _):
        for b in (0, 1):
            i = 2 * step + b

            @pl.when(i + 1 < n_s)
            def _(b=b, i=i):
                nxt = _fire_in(1 - b, i + 1)
                _compute(b)
                _out(b, i)
                for d in nxt:
                    d.wait()

            @pl.when((i < n_s) & (i + 1 >= n_s))
            def _(b=b, i=i):
                _compute(b)
                _out(b, i)
        return 0

    lax.fori_loop(0, (num_chunks // 16 + 2) // 2, _step, 0)
    pltpu.sync_copy(statb, statp_h.at[c * 16 + s])
    plsc.subcore_barrier()

    for cnt in counts:
        if cnt == 0:
            continue

        @pl.when(rows_per_sub == cnt)
        def _(cnt=cnt):
            off = 0
            while off < cnt:
                n = min(fl, cnt - off)
                src = pl.ds(base0 + off, n)
                dst = pl.ds(c * N + base0 + off, n)
                pltpu.sync_copy(h_acc.at[src], hsum_h.at[dst])
                pltpu.sync_copy(s_acc.at[src], ssum_h.at[dst])
                off += n


def _sc_edge_pass(row, col, Xs2, Xd2, Xu2, Ea2, N, E):
    C = 64
    H = 64
    mesh = plsc.VectorSubcoreMesh(core_axis_name="c", subcore_axis_name="s",
                                  num_cores=2, num_subcores=16)
    body = functools.partial(_sc_edge_body, N=N, E=E, C=C)
    buf_set = [
        pltpu.VMEM((C,), jnp.int32),
        pltpu.VMEM((C,), jnp.int32),
        pltpu.VMEM((C,), jnp.int32),
        pltpu.VMEM((C, H), F32),
        pltpu.VMEM((C, H), F32),
        pltpu.VMEM((C, H), F32),
        pltpu.VMEM((C, H), F32),
    ]
    k = pl.kernel(
        body,
        out_type=(
            jax.ShapeDtypeStruct((2 * E, H), F32),
            jax.ShapeDtypeStruct((2 * N, H), F32),
            jax.ShapeDtypeStruct((2 * N, H), F32),
            jax.ShapeDtypeStruct((32, 2, H), F32),
        ),
        mesh=mesh,
        compiler_params=pltpu.CompilerParams(use_tc_tiling_on_sc=False),
        scratch_types=[
            pltpu.VMEM_SHARED((N, H), F32),
            pltpu.VMEM_SHARED((N, H), F32),
            *buf_set,
            *buf_set,
            pltpu.VMEM((2, H), F32),
            pltpu.SemaphoreType.DMA,
        ],
    )
    return k(row, col, Xs2, Xd2, Xu2, Ea2)


def _edge_out_body(mL_ref, mR_ref, ea_ref, stat_ref, g_ref, b_ref,
                   out_ref, *, E):
    stat = stat_ref[...]
    halves = []
    for h, mr in ((0, mL_ref), (1, mR_ref)):
        ssum = jnp.sum(stat[h * 16:(h + 1) * 16, 0, :], axis=0)
        ssq = jnp.sum(stat[h * 16:(h + 1) * 16, 1, :], axis=0)
        mean = ssum / E
        var = ssq / E - mean * mean
        rstd = lax.rsqrt(var + 1e-5)
        z = (mr[...] - mean) * (rstd * g_ref[h]) + b_ref[h]
        halves.append(_softplus(z))
    out_ref[...] = ea_ref[...] + jnp.concatenate(halves, axis=1)


def _edge_out(m2, edge_attr, statp, g2, b2, E, block_rows):
    H = m2.shape[1]
    n = E // block_rows
    body = functools.partial(_edge_out_body, E=E)
    return pl.pallas_call(
        body,
        grid=(n,),
        in_specs=[
            pl.BlockSpec((block_rows, H), lambda i: (i, 0)),
            pl.BlockSpec((block_rows, H), lambda i, _n=n: (_n + i, 0)),
            pl.BlockSpec((block_rows, 2 * H), lambda i: (i, 0)),
            pl.BlockSpec((32, 2, H), lambda i: (0, 0, 0)),
            pl.BlockSpec((2, H), lambda i: (0, 0)),
            pl.BlockSpec((2, H), lambda i: (0, 0)),
        ],
        out_specs=pl.BlockSpec((block_rows, 2 * H), lambda i: (i, 0)),
        out_shape=jax.ShapeDtypeStruct((E, 2 * H), F32),
    )(m2, m2, edge_attr, statp, g2, b2)


def _node_out_body(x_ref, h2_ref, s2_ref, w_ref, b_ref, g_ref, bb_ref,
                   out_ref):
    hs = jnp.concatenate([h2_ref[0], h2_ref[1]], axis=1)
    ss = jnp.concatenate([s2_ref[0], s2_ref[1]], axis=1)
    hn = hs / (ss + 1e-6)
    xb = x_ref[...]
    u = lax.dot_general(xb, w_ref[...], (((1,), (1,)), ((), ())),
                        preferred_element_type=F32) + b_ref[...] + hn
    mu = jnp.mean(u, axis=0, keepdims=True)
    d = u - mu
    v = jnp.mean(d * d, axis=0, keepdims=True)
    z = d * lax.rsqrt(v + 1e-5) * g_ref[...] + bb_ref[...]
    out_ref[...] = xb + _softplus(z)


def _node_out(x, h2, s2, W_su, b_su, g, b):
    N, D = x.shape
    return pl.pallas_call(
        _node_out_body,
        out_shape=jax.ShapeDtypeStruct((N, D), F32),
    )(x, h2.reshape(2, N, D // 2), s2.reshape(2, N, D // 2),
      W_su, b_su.reshape(1, D), g.reshape(1, D), b.reshape(1, D))


def kernel(x, edge_index, edge_attr, W_sg, b_sg, W_dg, b_dg, W_eg, b_eg,
           W_su, b_su, W_du, b_du, bn_e_g, bn_e_b, bn_n_g, bn_n_b):
    N, D = x.shape
    E = edge_attr.shape[0]
    H = D // 2

    row = edge_index[0]
    col = edge_index[1]

    nb = _pick_block(N, 1000)
    eb = _pick_block(E, 2000)
    Xs2 = _project_halved(x, W_sg, b_sg, nb).reshape(2 * N, H)
    Xd2 = _project_halved(x, W_dg, b_dg, nb).reshape(2 * N, H)
    Xu2 = _project_halved(x, W_du, b_du, nb).reshape(2 * N, H)
    Ea2 = _project_halved(edge_attr, W_eg, b_eg, eb).reshape(2 * E, H)

    m2, h2, s2, statp = _sc_edge_pass(row, col, Xs2, Xd2, Xu2, Ea2, N, E)

    y_new = _edge_out(m2, edge_attr, statp,
                      bn_e_g.reshape(2, H), bn_e_b.reshape(2, H), E, eb)
    x_new = _node_out(x, h2, s2, W_su, b_su, bn_n_g, bn_n_b)
    return (x_new, y_new)


# speedup vs baseline: 1.0578x; 1.0578x over previous
import functools

import jax
import jax.numpy as jnp
from jax import lax
from jax.experimental import pallas as pl
from jax.experimental.pallas import tpu as pltpu
from jax.experimental.pallas import tpu_sc as plsc

F32 = jnp.float32


def _pick_block(n, target):
    d = min(n, target)
    while n % d:
        d -= 1
    return d


def _softplus(z):
    return jnp.maximum(z, 0.0) + jnp.log1p(jnp.exp(-jnp.abs(z)))


def _proj_body(x_ref, w_ref, b_ref, out_ref):
    xb = x_ref[...]
    for h in range(2):
        out_ref[h] = (
            lax.dot_general(xb, w_ref[h], (((1,), (1,)), ((), ())),
                            preferred_element_type=F32)
            + b_ref[h]
        )


def _project_halved(a, W, b, block_rows):
    R, D = a.shape
    H = D // 2
    grid = (R // block_rows,)
    W2 = W.reshape(2, H, D)
    b2 = b.reshape(2, H)
    return pl.pallas_call(
        _proj_body,
        grid=grid,
        in_specs=[
            pl.BlockSpec((block_rows, D), lambda i: (i, 0)),
            pl.BlockSpec((2, H, D), lambda i: (0, 0, 0)),
            pl.BlockSpec((2, H), lambda i: (0, 0)),
        ],
        out_specs=pl.BlockSpec((2, block_rows, H), lambda i: (0, i, 0)),
        out_shape=jax.ShapeDtypeStruct((2, R, H), F32),
    )(a, W2, b2)


def _sc_edge_body(rc_h, xs_h, xd_h, xu_h, ea_h,
                  m_h, hsum_h, ssum_h, statp_h,
                  h_acc, s_acc,
                  idxb0, xsb0, xdb0, xub0, eab0,
                  idxb1, xsb1, xdb1, xub1, eab1,
                  statb, gsem0,
                  *, N, E, C):
    idxb = (idxb0, idxb1)
    xsb = (xsb0, xsb1)
    xdb = (xdb0, xdb1)
    xub = (xub0, xub1)
    eab = (eab0, eab1)
    mb = (xdb0, xdb1)
    c = lax.axis_index("c")
    s = lax.axis_index("s")
    num_chunks = E // C

    q, r = divmod(N // 8, 16)
    counts = sorted({8 * q} | ({8 * (q + 1)} if r else set()))
    base0 = 8 * (q * s + jnp.minimum(s, r))

    def _zero_row(e, _):
        for j in range(4):
            xdb0[e, pl.ds(j * 16, 16)] = jnp.zeros((16,), F32)
        return 0
    lax.fori_loop(0, C, _zero_row, 0)
    for j in range(4):
        statb[0, pl.ds(j * 16, 16)] = jnp.zeros((16,), F32)
        statb[1, pl.ds(j * 16, 16)] = jnp.zeros((16,), F32)
    rows_per_sub = 8 * (q + (s < r).astype(jnp.int32))
    fl = min(120, C)
    for cnt in counts:
        if cnt == 0:
            continue

        @pl.when(rows_per_sub == cnt)
        def _(cnt=cnt):
            off = 0
            while off < cnt:
                n = min(fl, cnt - off)
                pltpu.sync_copy(xdb0.at[pl.ds(0, n)],
                                h_acc.at[pl.ds(base0 + off, n)])
                pltpu.sync_copy(xdb0.at[pl.ds(0, n)],
                                s_acc.at[pl.ds(base0 + off, n)])
                off += n
    plsc.subcore_barrier()

    goff = c * N
    n_s = (num_chunks - s + 15) // 16

    def _fire_in(b, i):
        chunk = s + i * 16
        base = chunk * C
        pltpu.sync_copy(rc_h.at[chunk], idxb[b].at[pl.ds(0, 2)])
        for j in range(C // 16):
            sl = pl.ds(j * 16, 16)
            idxb[b][0, sl] = idxb[b][0, sl] + goff
            idxb[b][2, sl] = idxb[b][1, sl] + goff
        return (
            pltpu.async_copy(xs_h.at[idxb[b].at[0]], xsb[b], gsem0),
            pltpu.async_copy(xd_h.at[idxb[b].at[2]], xdb[b], gsem0),
            pltpu.async_copy(xu_h.at[idxb[b].at[2]], xub[b], gsem0),
            pltpu.async_copy(ea_h.at[pl.ds(c * E + base, C)], eab[b], gsem0),
        )

    def _compute(b):
        def _edge(e, _):
            for j in range(4):
                sl = pl.ds(j * 16, 16)
                mv = xsb[b][e, sl] + xdb[b][e, sl] + eab[b][e, sl]
                mb[b][e, sl] = mv
                statb[0, sl] += mv
                statb[1, sl] += mv * mv
                sg = 1.0 / (1.0 + jnp.exp(-mv))
                xsb[b][e, sl] = sg
                xub[b][e, sl] = xub[b][e, sl] * sg
            return 0
        lax.fori_loop(0, C, _edge, 0)

    def _out(b, i):
        base = (s + i * 16) * C
        pltpu.sync_copy(mb[b], m_h.at[pl.ds(c * E + base, C)])
        pltpu.sync_copy(xsb[b], s_acc.at[idxb[b].at[1]], add=True)
        pltpu.sync_copy(xub[b], h_acc.at[idxb[b].at[1]], add=True)

    @pl.when(n_s > 0)
    def _():
        ds0 = _fire_in(0, 0)
        for d in ds0:
            d.wait()

    def _step(step, _):
        for b in (0, 1):
            i = 2 * step + b

            @pl.when(i + 1 < n_s)
            def _(b=b, i=i):
                nxt = _fire_in(1 - b, i + 1)
                _compute(b)
                _out(b, i)
                for d in nxt:
                    d.wait()

            @pl.when((i < n_s) & (i + 1 >= n_s))
            def _(b=b, i=i):
                _compute(b)
                _out(b, i)
        return 0

    lax.fori_loop(0, (num_chunks // 16 + 2) // 2, _step, 0)
    pltpu.sync_copy(statb, statp_h.at[c * 16 + s])
    plsc.subcore_barrier()

    for cnt in counts:
        if cnt == 0:
            continue

        @pl.when(rows_per_sub == cnt)
        def _(cnt=cnt):
            off = 0
            while off < cnt:
                n = min(fl, cnt - off)
                src = pl.ds(base0 + off, n)
                dst = pl.ds(c * N + base0 + off, n)
                pltpu.sync_copy(h_acc.at[src], hsum_h.at[dst])
                pltpu.sync_copy(s_acc.at[src], ssum_h.at[dst])
                off += n


def _sc_edge_pass(rc, Xs2, Xd2, Xu2, Ea2, N, E, C):
    H = 64
    mesh = plsc.VectorSubcoreMesh(core_axis_name="c", subcore_axis_name="s",
                                  num_cores=2, num_subcores=16)
    body = functools.partial(_sc_edge_body, N=N, E=E, C=C)
    H = 64
    buf_set = [
        pltpu.VMEM((3, C), jnp.int32),
        pltpu.VMEM((C, H), F32),
        pltpu.VMEM((C, H), F32),
        pltpu.VMEM((C, H), F32),
        pltpu.VMEM((C, H), F32),
    ]
    k = pl.kernel(
        body,
        out_type=(
            jax.ShapeDtypeStruct((2 * E, H), F32),
            jax.ShapeDtypeStruct((2 * N, H), F32),
            jax.ShapeDtypeStruct((2 * N, H), F32),
            jax.ShapeDtypeStruct((32, 2, H), F32),
        ),
        mesh=mesh,
        compiler_params=pltpu.CompilerParams(use_tc_tiling_on_sc=False),
        scratch_types=[
            pltpu.VMEM_SHARED((N, H), F32),
            pltpu.VMEM_SHARED((N, H), F32),
            *buf_set,
            *buf_set,
            pltpu.VMEM((2, H), F32),
            pltpu.SemaphoreType.DMA,
        ],
    )
    return k(rc, Xs2, Xd2, Xu2, Ea2)


def _edge_out_body(mL_ref, mR_ref, ea_ref, stat_ref, g_ref, b_ref,
                   out_ref, *, E):
    stat = stat_ref[...]
    halves = []
    for h, mr in ((0, mL_ref), (1, mR_ref)):
        ssum = jnp.sum(stat[h * 16:(h + 1) * 16, 0, :], axis=0)
        ssq = jnp.sum(stat[h * 16:(h + 1) * 16, 1, :], axis=0)
        mean = ssum / E
        var = ssq / E - mean * mean
        rstd = lax.rsqrt(var + 1e-5)
        z = (mr[...] - mean) * (rstd * g_ref[h]) + b_ref[h]
        halves.append(_softplus(z))
    out_ref[...] = ea_ref[...] + jnp.concatenate(halves, axis=1)


def _edge_out(m2, edge_attr, statp, g2, b2, E, block_rows):
    H = m2.shape[1]
    n = E // block_rows
    body = functools.partial(_edge_out_body, E=E)
    return pl.pallas_call(
        body,
        grid=(n,),
        in_specs=[
            pl.BlockSpec((block_rows, H), lambda i: (i, 0)),
            pl.BlockSpec((block_rows, H), lambda i, _n=n: (_n + i, 0)),
            pl.BlockSpec((block_rows, 2 * H), lambda i: (i, 0)),
            pl.BlockSpec((32, 2, H), lambda i: (0, 0, 0)),
            pl.BlockSpec((2, H), lambda i: (0, 0)),
            pl.BlockSpec((2, H), lambda i: (0, 0)),
        ],
        out_specs=pl.BlockSpec((block_rows, 2 * H), lambda i: (i, 0)),
        out_shape=jax.ShapeDtypeStruct((E, 2 * H), F32),
    )(m2, m2, edge_attr, statp, g2, b2)


def _node_out_body(x_ref, h2_ref, s2_ref, w_ref, b_ref, g_ref, bb_ref,
                   out_ref):
    hs = jnp.concatenate([h2_ref[0], h2_ref[1]], axis=1)
    ss = jnp.concatenate([s2_ref[0], s2_ref[1]], axis=1)
    hn = hs / (ss + 1e-6)
    xb = x_ref[...]
    u = lax.dot_general(xb, w_ref[...], (((1,), (1,)), ((), ())),
                        preferred_element_type=F32) + b_ref[...] + hn
    mu = jnp.mean(u, axis=0, keepdims=True)
    d = u - mu
    v = jnp.mean(d * d, axis=0, keepdims=True)
    z = d * lax.rsqrt(v + 1e-5) * g_ref[...] + bb_ref[...]
    out_ref[...] = xb + _softplus(z)


def _node_out(x, h2, s2, W_su, b_su, g, b):
    N, D = x.shape
    return pl.pallas_call(
        _node_out_body,
        out_shape=jax.ShapeDtypeStruct((N, D), F32),
    )(x, h2.reshape(2, N, D // 2), s2.reshape(2, N, D // 2),
      W_su, b_su.reshape(1, D), g.reshape(1, D), b.reshape(1, D))


def kernel(x, edge_index, edge_attr, W_sg, b_sg, W_dg, b_dg, W_eg, b_eg,
           W_su, b_su, W_du, b_du, bn_e_g, bn_e_b, bn_n_g, bn_n_b):
    N, D = x.shape
    E = edge_attr.shape[0]
    H = D // 2

    row = edge_index[0]
    col = edge_index[1]

    nb = _pick_block(N, 1000)
    eb = _pick_block(E, 2000)
    Xs2 = _project_halved(x, W_sg, b_sg, nb).reshape(2 * N, H)
    Xd2 = _project_halved(x, W_dg, b_dg, nb).reshape(2 * N, H)
    Xu2 = _project_halved(x, W_du, b_du, nb).reshape(2 * N, H)
    Ea2 = _project_halved(edge_attr, W_eg, b_eg, eb).reshape(2 * E, H)

    C = 80
    rc = jnp.stack([row.reshape(-1, C), col.reshape(-1, C)], axis=1)
    m2, h2, s2, statp = _sc_edge_pass(rc, Xs2, Xd2, Xu2, Ea2, N, E, C)

    y_new = _edge_out(m2, edge_attr, statp,
                      bn_e_g.reshape(2, H), bn_e_b.reshape(2, H), E, eb)
    x_new = _node_out(x, h2, s2, W_su, b_su, bn_n_g, bn_n_b)
    return (x_new, y_new)


# speedup vs baseline: 1.4771x; 1.3964x over previous
import functools

import jax
import jax.numpy as jnp
from jax import lax
from jax.experimental import pallas as pl
from jax.experimental.pallas import tpu as pltpu
from jax.experimental.pallas import tpu_sc as plsc

F32 = jnp.float32


def _pick_block(n, target):
    d = min(n, target)
    while n % d:
        d -= 1
    return d


def _softplus(z):
    return jnp.maximum(z, 0.0) + jnp.log1p(jnp.exp(-jnp.abs(z)))


def _proj_body(x_ref, w_ref, b_ref, out_ref):
    xb = x_ref[...]
    for h in range(2):
        out_ref[h] = (
            lax.dot_general(xb, w_ref[h], (((1,), (1,)), ((), ())),
                            preferred_element_type=F32)
            + b_ref[h]
        )


def _project_halved(a, W, b, block_rows):
    R, D = a.shape
    H = D // 2
    grid = (R // block_rows,)
    W2 = W.reshape(2, H, D)
    b2 = b.reshape(2, H)
    return pl.pallas_call(
        _proj_body,
        grid=grid,
        in_specs=[
            pl.BlockSpec((block_rows, D), lambda i: (i, 0)),
            pl.BlockSpec((2, H, D), lambda i: (0, 0, 0)),
            pl.BlockSpec((2, H), lambda i: (0, 0)),
        ],
        out_specs=pl.BlockSpec((2, block_rows, H), lambda i: (0, i, 0)),
        out_shape=jax.ShapeDtypeStruct((2, R, H), F32),
    )(a, W2, b2)


def _sc_edge_body(rc_h, xs_h, xd_h, xu_h, ea_h,
                  m_h, hsum_h, ssum_h, statp_h,
                  h_acc, s_acc,
                  idxb0, xsb0, xdb0, xub0, eab0,
                  idxb1, xsb1, xdb1, xub1, eab1,
                  statb, gsem0, msem0, msem1,
                  *, N, E, C):
    idxb = (idxb0, idxb1)
    xsb = (xsb0, xsb1)
    xdb = (xdb0, xdb1)
    xub = (xub0, xub1)
    eab = (eab0, eab1)
    mb = (xdb0, xdb1)
    msem = (msem0, msem1)
    c = lax.axis_index("c")
    s = lax.axis_index("s")
    num_chunks = E // C

    q, r = divmod(N // 8, 16)
    counts = sorted({8 * q} | ({8 * (q + 1)} if r else set()))
    base0 = 8 * (q * s + jnp.minimum(s, r))

    def _zero_row(e, _):
        for j in range(4):
            xdb0[e, pl.ds(j * 16, 16)] = jnp.zeros((16,), F32)
        return 0
    lax.fori_loop(0, C, _zero_row, 0)
    for j in range(4):
        statb[0, pl.ds(j * 16, 16)] = jnp.zeros((16,), F32)
        statb[1, pl.ds(j * 16, 16)] = jnp.zeros((16,), F32)
    rows_per_sub = 8 * (q + (s < r).astype(jnp.int32))
    fl = min(120, C)
    for cnt in counts:
        if cnt == 0:
            continue

        @pl.when(rows_per_sub == cnt)
        def _(cnt=cnt):
            off = 0
            while off < cnt:
                n = min(fl, cnt - off)
                pltpu.sync_copy(xdb0.at[pl.ds(0, n)],
                                h_acc.at[pl.ds(base0 + off, n)])
                pltpu.sync_copy(xdb0.at[pl.ds(0, n)],
                                s_acc.at[pl.ds(base0 + off, n)])
                off += n
    plsc.subcore_barrier()

    goff = c * N
    n_s = (num_chunks - s + 15) // 16

    def _fire_in(b, i):
        chunk = s + i * 16
        base = chunk * C
        pltpu.sync_copy(rc_h.at[chunk], idxb[b].at[pl.ds(0, 2)])
        for j in range(C // 16):
            sl = pl.ds(j * 16, 16)
            idxb[b][0, sl] = idxb[b][0, sl] + goff
            idxb[b][2, sl] = idxb[b][1, sl] + goff
        return (
            pltpu.async_copy(xs_h.at[idxb[b].at[0]], xsb[b], gsem0),
            pltpu.async_copy(xd_h.at[idxb[b].at[2]], xdb[b], gsem0),
            pltpu.async_copy(xu_h.at[idxb[b].at[2]], xub[b], gsem0),
            pltpu.async_copy(ea_h.at[pl.ds(c * E + base, C)], eab[b], gsem0),
        )

    def _compute(b):
        def _edge(e, st):
            out = []
            for j in range(4):
                sl = pl.ds(j * 16, 16)
                mv = xsb[b][e, sl] + xdb[b][e, sl] + eab[b][e, sl]
                mb[b][e, sl] = mv
                out.append(st[2 * j] + mv)
                out.append(st[2 * j + 1] + mv * mv)
                sg = 1.0 / (1.0 + jnp.exp(-mv))
                xsb[b][e, sl] = sg
                xub[b][e, sl] = xub[b][e, sl] * sg
            return tuple(out)
        st0 = tuple(statb[i, pl.ds(j * 16, 16)]
                    for j in range(4) for i in range(2))
        stf = lax.fori_loop(0, C, _edge, st0)
        for j in range(4):
            statb[0, pl.ds(j * 16, 16)] = stf[2 * j]
            statb[1, pl.ds(j * 16, 16)] = stf[2 * j + 1]

    def _out(b, i):
        base = (s + i * 16) * C
        pltpu.async_copy(mb[b], m_h.at[pl.ds(c * E + base, C)], msem[b])
        pltpu.sync_copy(xsb[b], s_acc.at[idxb[b].at[1]], add=True)
        pltpu.sync_copy(xub[b], h_acc.at[idxb[b].at[1]], add=True)

    def _drain_m(b):
        pltpu.make_async_copy(mb[b], m_h.at[pl.ds(0, C)], msem[b]).wait()

    @pl.when(n_s > 0)
    def _():
        ds0 = _fire_in(0, 0)
        for d in ds0:
            d.wait()

    def _step(step, _):
        for b in (0, 1):
            i = 2 * step + b

            @pl.when(i + 1 < n_s)
            def _(b=b, i=i):
                nxt = _fire_in(1 - b, i + 1)

                @pl.when(i >= 2)
                def _(b=b):
                    _drain_m(b)
                _compute(b)
                _out(b, i)
                for d in nxt:
                    d.wait()

            @pl.when((i < n_s) & (i + 1 >= n_s))
            def _(b=b, i=i):
                @pl.when(i >= 2)
                def _(b=b):
                    _drain_m(b)
                _compute(b)
                _out(b, i)
        return 0

    lax.fori_loop(0, (num_chunks // 16 + 2) // 2, _step, 0)

    @pl.when(n_s >= 1)
    def _():
        _drain_m(0)

    @pl.when(n_s >= 2)
    def _():
        _drain_m(1)
    pltpu.sync_copy(statb, statp_h.at[c * 16 + s])
    plsc.subcore_barrier()

    for cnt in counts:
        if cnt == 0:
            continue

        @pl.when(rows_per_sub == cnt)
        def _(cnt=cnt):
            off = 0
            while off < cnt:
                n = min(fl, cnt - off)
                src = pl.ds(base0 + off, n)
                dst = pl.ds(c * N + base0 + off, n)
                pltpu.sync_copy(h_acc.at[src], hsum_h.at[dst])
                pltpu.sync_copy(s_acc.at[src], ssum_h.at[dst])
                off += n


def _sc_edge_pass(rc, Xs2, Xd2, Xu2, Ea2, N, E, C):
    H = 64
    mesh = plsc.VectorSubcoreMesh(core_axis_name="c", subcore_axis_name="s",
                                  num_cores=2, num_subcores=16)
    body = functools.partial(_sc_edge_body, N=N, E=E, C=C)
    H = 64
    buf_set = [
        pltpu.VMEM((3, C), jnp.int32),
        pltpu.VMEM((C, H), F32),
        pltpu.VMEM((C, H), F32),
        pltpu.VMEM((C, H), F32),
        pltpu.VMEM((C, H), F32),
    ]
    k = pl.kernel(
        body,
        out_type=(
            jax.ShapeDtypeStruct((2 * E, H), F32),
            jax.ShapeDtypeStruct((2 * N, H), F32),
            jax.ShapeDtypeStruct((2 * N, H), F32),
            jax.ShapeDtypeStruct((32, 2, H), F32),
        ),
        mesh=mesh,
        compiler_params=pltpu.CompilerParams(use_tc_tiling_on_sc=False),
        scratch_types=[
            pltpu.VMEM_SHARED((N, H), F32),
            pltpu.VMEM_SHARED((N, H), F32),
            *buf_set,
            *buf_set,
            pltpu.VMEM((2, H), F32),
            pltpu.SemaphoreType.DMA,
            pltpu.SemaphoreType.DMA,
            pltpu.SemaphoreType.DMA,
        ],
    )
    return k(rc, Xs2, Xd2, Xu2, Ea2)


def _edge_out_body(mL_ref, mR_ref, ea_ref, stat_ref, g_ref, b_ref,
                   out_ref, *, E):
    stat = stat_ref[...]
    halves = []
    for h, mr in ((0, mL_ref), (1, mR_ref)):
        ssum = jnp.sum(stat[h * 16:(h + 1) * 16, 0, :], axis=0)
        ssq = jnp.sum(stat[h * 16:(h + 1) * 16, 1, :], axis=0)
        mean = ssum / E
        var = ssq / E - mean * mean
        rstd = lax.rsqrt(var + 1e-5)
        z = (mr[...] - mean) * (rstd * g_ref[h]) + b_ref[h]
        halves.append(_softplus(z))
    out_ref[...] = ea_ref[...] + jnp.concatenate(halves, axis=1)


def _edge_out(m2, edge_attr, statp, g2, b2, E, block_rows):
    H = m2.shape[1]
    n = E // block_rows
    body = functools.partial(_edge_out_body, E=E)
    return pl.pallas_call(
        body,
        grid=(n,),
        in_specs=[
            pl.BlockSpec((block_rows, H), lambda i: (i, 0)),
            pl.BlockSpec((block_rows, H), lambda i, _n=n: (_n + i, 0)),
            pl.BlockSpec((block_rows, 2 * H), lambda i: (i, 0)),
            pl.BlockSpec((32, 2, H), lambda i: (0, 0, 0)),
            pl.BlockSpec((2, H), lambda i: (0, 0)),
            pl.BlockSpec((2, H), lambda i: (0, 0)),
        ],
        out_specs=pl.BlockSpec((block_rows, 2 * H), lambda i: (i, 0)),
        out_shape=jax.ShapeDtypeStruct((E, 2 * H), F32),
    )(m2, m2, edge_attr, statp, g2, b2)


def _node_out_body(x_ref, h2_ref, s2_ref, w_ref, b_ref, g_ref, bb_ref,
                   out_ref):
    hs = jnp.concatenate([h2_ref[0], h2_ref[1]], axis=1)
    ss = jnp.concatenate([s2_ref[0], s2_ref[1]], axis=1)
    hn = hs / (ss + 1e-6)
    xb = x_ref[...]
    u = lax.dot_general(xb, w_ref[...], (((1,), (1,)), ((), ())),
                        preferred_element_type=F32) + b_ref[...] + hn
    mu = jnp.mean(u, axis=0, keepdims=True)
    d = u - mu
    v = jnp.mean(d * d, axis=0, keepdims=True)
    z = d * lax.rsqrt(v + 1e-5) * g_ref[...] + bb_ref[...]
    out_ref[...] = xb + _softplus(z)


def _node_out(x, h2, s2, W_su, b_su, g, b):
    N, D = x.shape
    return pl.pallas_call(
        _node_out_body,
        out_shape=jax.ShapeDtypeStruct((N, D), F32),
    )(x, h2.reshape(2, N, D // 2), s2.reshape(2, N, D // 2),
      W_su, b_su.reshape(1, D), g.reshape(1, D), b.reshape(1, D))


def kernel(x, edge_index, edge_attr, W_sg, b_sg, W_dg, b_dg, W_eg, b_eg,
           W_su, b_su, W_du, b_du, bn_e_g, bn_e_b, bn_n_g, bn_n_b):
    N, D = x.shape
    E = edge_attr.shape[0]
    H = D // 2

    row = edge_index[0]
    col = edge_index[1]

    nb = _pick_block(N, 1000)
    eb = _pick_block(E, 2000)
    Xs2 = _project_halved(x, W_sg, b_sg, nb).reshape(2 * N, H)
    Xd2 = _project_halved(x, W_dg, b_dg, nb).reshape(2 * N, H)
    Xu2 = _project_halved(x, W_du, b_du, nb).reshape(2 * N, H)
    Ea2 = _project_halved(edge_attr, W_eg, b_eg, eb).reshape(2 * E, H)

    C = 80
    rc = jnp.stack([row.reshape(-1, C), col.reshape(-1, C)], axis=1)
    m2, h2, s2, statp = _sc_edge_pass(rc, Xs2, Xd2, Xu2, Ea2, N, E, C)

    y_new = _edge_out(m2, edge_attr, statp,
                      bn_e_g.reshape(2, H), bn_e_b.reshape(2, H), E, eb)
    x_new = _node_out(x, h2, s2, W_su, b_su, bn_n_g, bn_n_b)
    return (x_new, y_new)


# speedup vs baseline: 1.4798x; 1.0018x over previous
import functools

import jax
import jax.numpy as jnp
from jax import lax
from jax.experimental import pallas as pl
from jax.experimental.pallas import tpu as pltpu
from jax.experimental.pallas import tpu_sc as plsc

F32 = jnp.float32


def _pick_block(n, target):
    d = min(n, target)
    while n % d:
        d -= 1
    return d


def _softplus(z):
    return jnp.maximum(z, 0.0) + jnp.log1p(jnp.exp(-jnp.abs(z)))


def _proj_body(x_ref, w_ref, b_ref, out_ref):
    xb = x_ref[...]
    for h in range(2):
        out_ref[h] = (
            lax.dot_general(xb, w_ref[h], (((1,), (1,)), ((), ())),
                            preferred_element_type=F32)
            + b_ref[h]
        )


def _project_halved(a, W, b, block_rows):
    R, D = a.shape
    H = D // 2
    grid = (R // block_rows,)
    W2 = W.reshape(2, H, D)
    b2 = b.reshape(2, H)
    return pl.pallas_call(
        _proj_body,
        grid=grid,
        in_specs=[
            pl.BlockSpec((block_rows, D), lambda i: (i, 0)),
            pl.BlockSpec((2, H, D), lambda i: (0, 0, 0)),
            pl.BlockSpec((2, H), lambda i: (0, 0)),
        ],
        out_specs=pl.BlockSpec((2, block_rows, H), lambda i: (0, i, 0)),
        out_shape=jax.ShapeDtypeStruct((2, R, H), F32),
    )(a, W2, b2)


def _sc_edge_body(rc_h, xs_h, xd_h, xu_h, ea_h,
                  m_h, hsum_h, ssum_h, statp_h,
                  h_acc, s_acc,
                  idxb0, xsb0, xdb0, xub0, eab0,
                  idxb1, xsb1, xdb1, xub1, eab1,
                  statb, gsem0, msem0, msem1, isem0, isem1,
                  *, N, E, C):
    idxb = (idxb0, idxb1)
    xsb = (xsb0, xsb1)
    xdb = (xdb0, xdb1)
    xub = (xub0, xub1)
    eab = (eab0, eab1)
    mb = (xdb0, xdb1)
    msem = (msem0, msem1)
    isem = (isem0, isem1)
    c = lax.axis_index("c")
    s = lax.axis_index("s")
    num_chunks = E // C

    q, r = divmod(N // 8, 16)
    counts = sorted({8 * q} | ({8 * (q + 1)} if r else set()))
    base0 = 8 * (q * s + jnp.minimum(s, r))

    def _zero_row(e, _):
        for j in range(4):
            xdb0[e, pl.ds(j * 16, 16)] = jnp.zeros((16,), F32)
        return 0
    lax.fori_loop(0, C, _zero_row, 0)
    for j in range(4):
        statb[0, pl.ds(j * 16, 16)] = jnp.zeros((16,), F32)
        statb[1, pl.ds(j * 16, 16)] = jnp.zeros((16,), F32)
    rows_per_sub = 8 * (q + (s < r).astype(jnp.int32))
    fl = min(120, C)
    for cnt in counts:
        if cnt == 0:
            continue

        @pl.when(rows_per_sub == cnt)
        def _(cnt=cnt):
            off = 0
            while off < cnt:
                n = min(fl, cnt - off)
                pltpu.sync_copy(xdb0.at[pl.ds(0, n)],
                                h_acc.at[pl.ds(base0 + off, n)])
                pltpu.sync_copy(xdb0.at[pl.ds(0, n)],
                                s_acc.at[pl.ds(base0 + off, n)])
                off += n
    plsc.subcore_barrier()

    goff = c * N
    n_s = (num_chunks - s + 15) // 16

    def _load_idx(b, i):
        chunk = s + i * 16
        pltpu.async_copy(rc_h.at[chunk], idxb[b].at[pl.ds(0, 2)], isem[b])

    def _drain_idx(b):
        pltpu.make_async_copy(rc_h.at[0], idxb[b].at[pl.ds(0, 2)],
                              isem[b]).wait()

    def _fire_in(b, i):
        base = (s + i * 16) * C
        for j in range(C // 16):
            sl = pl.ds(j * 16, 16)
            idxb[b][0, sl] = idxb[b][0, sl] + goff
            idxb[b][2, sl] = idxb[b][1, sl] + goff
        return (
            pltpu.async_copy(xs_h.at[idxb[b].at[0]], xsb[b], gsem0),
            pltpu.async_copy(xd_h.at[idxb[b].at[2]], xdb[b], gsem0),
            pltpu.async_copy(xu_h.at[idxb[b].at[2]], xub[b], gsem0),
            pltpu.async_copy(ea_h.at[pl.ds(c * E + base, C)], eab[b], gsem0),
        )

    def _compute(b):
        def _edge(e, st):
            out = []
            for j in range(4):
                sl = pl.ds(j * 16, 16)
                mv = xsb[b][e, sl] + xdb[b][e, sl] + eab[b][e, sl]
                mb[b][e, sl] = mv
                out.append(st[2 * j] + mv)
                out.append(st[2 * j + 1] + mv * mv)
                sg = 1.0 / (1.0 + jnp.exp(-mv))
                xsb[b][e, sl] = sg
                xub[b][e, sl] = xub[b][e, sl] * sg
            return tuple(out)
        st0 = tuple(statb[i, pl.ds(j * 16, 16)]
                    for j in range(4) for i in range(2))
        stf = lax.fori_loop(0, C, _edge, st0)
        for j in range(4):
            statb[0, pl.ds(j * 16, 16)] = stf[2 * j]
            statb[1, pl.ds(j * 16, 16)] = stf[2 * j + 1]

    def _out(b, i):
        base = (s + i * 16) * C
        pltpu.async_copy(mb[b], m_h.at[pl.ds(c * E + base, C)], msem[b])
        pltpu.sync_copy(xsb[b], s_acc.at[idxb[b].at[1]], add=True)
        pltpu.sync_copy(xub[b], h_acc.at[idxb[b].at[1]], add=True)

    def _drain_m(b):
        pltpu.make_async_copy(mb[b], m_h.at[pl.ds(0, C)], msem[b]).wait()

    @pl.when(n_s > 0)
    def _():
        pltpu.sync_copy(rc_h.at[s], idxb[0].at[pl.ds(0, 2)])
        ds0 = _fire_in(0, 0)
        for d in ds0:
            d.wait()

    @pl.when(n_s > 1)
    def _():
        _load_idx(1, 1)

    def _step(step, _):
        for b in (0, 1):
            i = 2 * step + b

            @pl.when(i + 1 < n_s)
            def _(b=b, i=i):
                _drain_idx(1 - b)
                nxt = _fire_in(1 - b, i + 1)

                @pl.when(i >= 2)
                def _(b=b):
                    _drain_m(b)
                _compute(b)
                _out(b, i)

                @pl.when(i + 2 < n_s)
                def _(b=b, i=i):
                    _load_idx(b, i + 2)
                for d in nxt:
                    d.wait()

            @pl.when((i < n_s) & (i + 1 >= n_s))
            def _(b=b, i=i):
                @pl.when(i >= 2)
                def _(b=b):
                    _drain_m(b)
                _compute(b)
                _out(b, i)
        return 0

    lax.fori_loop(0, (num_chunks // 16 + 2) // 2, _step, 0)

    @pl.when(n_s >= 1)
    def _():
        _drain_m(0)

    @pl.when(n_s >= 2)
    def _():
        _drain_m(1)
    pltpu.sync_copy(statb, statp_h.at[c * 16 + s])
    plsc.subcore_barrier()

    for cnt in counts:
        if cnt == 0:
            continue

        @pl.when(rows_per_sub == cnt)
        def _(cnt=cnt):
            off = 0
            while off < cnt:
                n = min(fl, cnt - off)
                src = pl.ds(base0 + off, n)
                dst = pl.ds(c * N + base0 + off, n)
                pltpu.sync_copy(h_acc.at[src], hsum_h.at[dst])
                pltpu.sync_copy(s_acc.at[src], ssum_h.at[dst])
                off += n


def _sc_edge_pass(rc, Xs2, Xd2, Xu2, Ea2, N, E, C):
    H = 64
    mesh = plsc.VectorSubcoreMesh(core_axis_name="c", subcore_axis_name="s",
                                  num_cores=2, num_subcores=16)
    body = functools.partial(_sc_edge_body, N=N, E=E, C=C)
    H = 64
    buf_set = [
        pltpu.VMEM((3, C), jnp.int32),
        pltpu.VMEM((C, H), F32),
        pltpu.VMEM((C, H), F32),
        pltpu.VMEM((C, H), F32),
        pltpu.VMEM((C, H), F32),
    ]
    k = pl.kernel(
        body,
        out_type=(
            jax.ShapeDtypeStruct((2 * E, H), F32),
            jax.ShapeDtypeStruct((2 * N, H), F32),
            jax.ShapeDtypeStruct((2 * N, H), F32),
            jax.ShapeDtypeStruct((32, 2, H), F32),
        ),
        mesh=mesh,
        compiler_params=pltpu.CompilerParams(use_tc_tiling_on_sc=False),
        scratch_types=[
            pltpu.VMEM_SHARED((N, H), F32),
            pltpu.VMEM_SHARED((N, H), F32),
            *buf_set,
            *buf_set,
            pltpu.VMEM((2, H), F32),
            pltpu.SemaphoreType.DMA,
            pltpu.SemaphoreType.DMA,
            pltpu.SemaphoreType.DMA,
            pltpu.SemaphoreType.DMA,
            pltpu.SemaphoreType.DMA,
        ],
    )
    return k(rc, Xs2, Xd2, Xu2, Ea2)


def _edge_out_body(mL_ref, mR_ref, ea_ref, stat_ref, g_ref, b_ref,
                   out_ref, *, E):
    stat = stat_ref[...]
    halves = []
    for h, mr in ((0, mL_ref), (1, mR_ref)):
        ssum = jnp.sum(stat[h * 16:(h + 1) * 16, 0, :], axis=0)
        ssq = jnp.sum(stat[h * 16:(h + 1) * 16, 1, :], axis=0)
        mean = ssum / E
        var = ssq / E - mean * mean
        rstd = lax.rsqrt(var + 1e-5)
        z = (mr[...] - mean) * (rstd * g_ref[h]) + b_ref[h]
        halves.append(_softplus(z))
    out_ref[...] = ea_ref[...] + jnp.concatenate(halves, axis=1)


def _edge_out(m2, edge_attr, statp, g2, b2, E, block_rows):
    H = m2.shape[1]
    n = E // block_rows
    body = functools.partial(_edge_out_body, E=E)
    return pl.pallas_call(
        body,
        grid=(n,),
        in_specs=[
            pl.BlockSpec((block_rows, H), lambda i: (i, 0)),
            pl.BlockSpec((block_rows, H), lambda i, _n=n: (_n + i, 0)),
            pl.BlockSpec((block_rows, 2 * H), lambda i: (i, 0)),
            pl.BlockSpec((32, 2, H), lambda i: (0, 0, 0)),
            pl.BlockSpec((2, H), lambda i: (0, 0)),
            pl.BlockSpec((2, H), lambda i: (0, 0)),
        ],
        out_specs=pl.BlockSpec((block_rows, 2 * H), lambda i: (i, 0)),
        out_shape=jax.ShapeDtypeStruct((E, 2 * H), F32),
    )(m2, m2, edge_attr, statp, g2, b2)


def _node_out_body(x_ref, h2_ref, s2_ref, w_ref, b_ref, g_ref, bb_ref,
                   out_ref):
    hs = jnp.concatenate([h2_ref[0], h2_ref[1]], axis=1)
    ss = jnp.concatenate([s2_ref[0], s2_ref[1]], axis=1)
    hn = hs / (ss + 1e-6)
    xb = x_ref[...]
    u = lax.dot_general(xb, w_ref[...], (((1,), (1,)), ((), ())),
                        preferred_element_type=F32) + b_ref[...] + hn
    mu = jnp.mean(u, axis=0, keepdims=True)
    d = u - mu
    v = jnp.mean(d * d, axis=0, keepdims=True)
    z = d * lax.rsqrt(v + 1e-5) * g_ref[...] + bb_ref[...]
    out_ref[...] = xb + _softplus(z)


def _node_out(x, h2, s2, W_su, b_su, g, b):
    N, D = x.shape
    return pl.pallas_call(
        _node_out_body,
        out_shape=jax.ShapeDtypeStruct((N, D), F32),
    )(x, h2.reshape(2, N, D // 2), s2.reshape(2, N, D // 2),
      W_su, b_su.reshape(1, D), g.reshape(1, D), b.reshape(1, D))


def kernel(x, edge_index, edge_attr, W_sg, b_sg, W_dg, b_dg, W_eg, b_eg,
           W_su, b_su, W_du, b_du, bn_e_g, bn_e_b, bn_n_g, bn_n_b):
    N, D = x.shape
    E = edge_attr.shape[0]
    H = D // 2

    row = edge_index[0]
    col = edge_index[1]

    nb = _pick_block(N, 1000)
    eb = _pick_block(E, 2000)
    Xs2 = _project_halved(x, W_sg, b_sg, nb).reshape(2 * N, H)
    Xd2 = _project_halved(x, W_dg, b_dg, nb).reshape(2 * N, H)
    Xu2 = _project_halved(x, W_du, b_du, nb).reshape(2 * N, H)
    Ea2 = _project_halved(edge_attr, W_eg, b_eg, eb).reshape(2 * E, H)

    C = 80
    rc = jnp.stack([row.reshape(-1, C), col.reshape(-1, C)], axis=1)
    m2, h2, s2, statp = _sc_edge_pass(rc, Xs2, Xd2, Xu2, Ea2, N, E, C)

    y_new = _edge_out(m2, edge_attr, statp,
                      bn_e_g.reshape(2, H), bn_e_b.reshape(2, H), E, eb)
    x_new = _node_out(x, h2, s2, W_su, b_su, bn_n_g, bn_n_b)
    return (x_new, y_new)
